# baseline (device time: 122792 ns/iter reference)
import os

import jax
import jax.numpy as jnp
from jax import lax
from jax.experimental import pallas as pl
from jax.experimental.pallas import tpu as pltpu

N_DEV = 8
N_SUB = 4
N_HALF = 2
_SKIP_COMPUTE = os.environ.get("ABLATE_COMPUTE", "0") == "1"
_SKIP_COMM = os.environ.get("ABLATE_COMM", "0") == "1"
_HALVES = 1 if os.environ.get("ABLATE_HALF", "0") == "1" else N_HALF


def kernel(x, w_mat, scale_x, scale_w):
    m_per, k = x.shape
    _, n_per = w_mat.shape
    m_sub = m_per // N_SUB
    m_half = m_sub // N_HALF

    x8 = x.astype(jnp.float8_e4m3fn)
    w8 = w_mat.astype(jnp.float8_e5m2)
    s = (scale_x.reshape(-1)[:1] * scale_w.reshape(-1)[:1]).astype(jnp.float32)

    def body(x_ref, w_ref, s_ref, out_ref, *scratch):
        bufs = scratch[0:N_SUB]
        sends = scratch[N_SUB:2 * N_SUB]
        recvs = scratch[2 * N_SUB:3 * N_SUB]

        my = lax.axis_index("i")

        def perm(r):
            return jnp.where(r < 4, r, 11 - r)

        my_r = perm(my)
        left = perm(lax.rem(my_r + N_DEV - 1, N_DEV))
        right = perm(lax.rem(my_r + 1, N_DEV))

        target = {0: right, 1: right, 2: left, 3: left}

        barrier_sem = pltpu.get_barrier_semaphore()
        for nbr in (left, right):
            pl.semaphore_signal(
                barrier_sem, inc=1,
                device_id=(nbr,), device_id_type=pl.DeviceIdType.MESH,
            )
        pl.semaphore_wait(barrier_sem, 2)

        def hop(si, h, half):
            rows = slice(half * m_half, (half + 1) * m_half)
            if h == 0:
                src = x_ref.at[pl.ds(si * m_sub + half * m_half, m_half), :]
            else:
                src = bufs[si].at[h, rows]
            return pltpu.make_async_remote_copy(
                src_ref=src,
                dst_ref=bufs[si].at[h + 1, rows],
                send_sem=sends[si].at[h, half],
                recv_sem=recvs[si].at[h, half],
                device_id=(target[si],),
                device_id_type=pl.DeviceIdType.MESH,
            )

        def store_own(si):
            if _SKIP_COMPUTE:
                return
            acc = lax.dot_general(
                x_ref[si * m_sub:(si + 1) * m_sub, :], w_ref[...],
                (((1,), (0,)), ((), ())),
                preferred_element_type=jnp.float32,
            )
            out_ref[pl.ds(my * m_per + si * m_sub, m_sub), :] = acc * s_ref[0]

        def store(si, h, origin):
            if _SKIP_COMPUTE:
                return
            acc = lax.dot_general(
                bufs[si][h], w_ref[...],
                (((1,), (0,)), ((), ())),
                preferred_element_type=jnp.float32,
            )
            out_ref[pl.ds(origin * m_per + si * m_sub, m_sub), :] = acc * s_ref[0]

        if not _SKIP_COMM:
            for half in range(_HALVES):
                for si in range(N_SUB):
                    hop(si, 0, half).start()
        for si in range(N_SUB):
            store_own(si)

        for h in range(N_DEV - 1):
            cw_origin = perm(lax.rem(my_r + N_DEV - 1 - h, N_DEV))
            ccw_origin = perm(lax.rem(my_r + 1 + h, N_DEV))
            for pair in ((0, 2), (1, 3)):
                if not _SKIP_COMM:
                    for half in range(_HALVES):
                        for si in pair:
                            hop(si, h, half).wait_recv()
                            if h < N_DEV - 2:
                                hop(si, h + 1, half).start()
                store(pair[0], h + 1, cw_origin)
                store(pair[1], h + 1, ccw_origin)

        if not _SKIP_COMM:
            for si in range(N_SUB):
                for h in range(N_DEV - 1):
                    for half in range(_HALVES):
                        hop(si, h, half).wait_send()

    comm = pltpu.VMEM((N_DEV, m_sub, k), jnp.float8_e4m3fn)
    sems = pltpu.SemaphoreType.DMA((N_DEV - 1, N_HALF))
    return pl.pallas_call(
        body,
        out_shape=jax.ShapeDtypeStruct((N_DEV * m_per, n_per), jnp.float32),
        in_specs=[
            pl.BlockSpec(memory_space=pltpu.VMEM),
            pl.BlockSpec(memory_space=pltpu.VMEM),
            pl.BlockSpec(memory_space=pltpu.SMEM),
        ],
        out_specs=pl.BlockSpec(memory_space=pltpu.VMEM),
        scratch_shapes=[comm] * N_SUB + [sems] * (2 * N_SUB),
        compiler_params=pltpu.CompilerParams(
            collective_id=0, vmem_limit_bytes=100 * 1024 * 1024),
    )(x8, w8, s)


# device time: 113671 ns/iter; 1.0802x vs baseline; 1.0802x over previous
import jax
import jax.numpy as jnp
from jax import lax
from jax.experimental import pallas as pl
from jax.experimental.pallas import tpu as pltpu

N_DEV = 8
N_SUB = 4
N_HALF = 2


def kernel(x, w_mat, scale_x, scale_w):
    m_per, k = x.shape
    _, n_per = w_mat.shape
    m_sub = m_per // N_SUB
    m_half = m_sub // N_HALF

    s = (scale_x.reshape(-1)[:1] * scale_w.reshape(-1)[:1]).astype(jnp.float32)

    def body(x_ref, w_ref, s_ref, out_ref, *scratch):
        bufs = scratch[0:N_SUB]
        sends = scratch[N_SUB:2 * N_SUB]
        recvs = scratch[2 * N_SUB:3 * N_SUB]
        w8_ref = scratch[3 * N_SUB]

        my = lax.axis_index("i")

        def perm(r):
            return jnp.where(r < 4, r, 11 - r)

        my_r = perm(my)
        left = perm(lax.rem(my_r + N_DEV - 1, N_DEV))
        right = perm(lax.rem(my_r + 1, N_DEV))

        target = {0: right, 1: right, 2: left, 3: left}

        barrier_sem = pltpu.get_barrier_semaphore()
        for nbr in (left, right):
            pl.semaphore_signal(
                barrier_sem, inc=1,
                device_id=(nbr,), device_id_type=pl.DeviceIdType.MESH,
            )
        pl.semaphore_wait(barrier_sem, 2)

        def hop(si, h, half):
            rows = slice(half * m_half, (half + 1) * m_half)
            return pltpu.make_async_remote_copy(
                src_ref=bufs[si].at[h, rows],
                dst_ref=bufs[si].at[h + 1, rows],
                send_sem=sends[si].at[h, half],
                recv_sem=recvs[si].at[h, half],
                device_id=(target[si],),
                device_id_type=pl.DeviceIdType.MESH,
            )

        def store(si, h, origin):
            acc = lax.dot_general(
                bufs[si][h], w8_ref[...],
                (((1,), (0,)), ((), ())),
                preferred_element_type=jnp.float32,
            )
            out_ref[pl.ds(origin * m_per + si * m_sub, m_sub), :] = acc * s_ref[0]

        for si in range(N_SUB):
            bufs[si][0] = x_ref[si * m_sub:(si + 1) * m_sub, :].astype(
                jnp.float8_e4m3fn)
        for half in range(N_HALF):
            for si in range(N_SUB):
                hop(si, 0, half).start()
        w8_ref[...] = w_ref[...].astype(jnp.float8_e5m2)
        for si in range(N_SUB):
            store(si, 0, my)

        for h in range(N_DEV - 1):
            cw_origin = perm(lax.rem(my_r + N_DEV - 1 - h, N_DEV))
            ccw_origin = perm(lax.rem(my_r + 1 + h, N_DEV))
            for pair in ((0, 2), (1, 3)):
                for half in range(N_HALF):
                    for si in pair:
                        hop(si, h, half).wait_recv()
                        if h < N_DEV - 2:
                            hop(si, h + 1, half).start()
                store(pair[0], h + 1, cw_origin)
                store(pair[1], h + 1, ccw_origin)

        for si in range(N_SUB):
            for h in range(N_DEV - 1):
                for half in range(N_HALF):
                    hop(si, h, half).wait_send()

    comm = pltpu.VMEM((N_DEV, m_sub, k), jnp.float8_e4m3fn)
    sems = pltpu.SemaphoreType.DMA((N_DEV - 1, N_HALF))
    return pl.pallas_call(
        body,
        out_shape=jax.ShapeDtypeStruct((N_DEV * m_per, n_per), jnp.float32),
        in_specs=[
            pl.BlockSpec(memory_space=pltpu.VMEM),
            pl.BlockSpec(memory_space=pltpu.VMEM),
            pl.BlockSpec(memory_space=pltpu.SMEM),
        ],
        out_specs=pl.BlockSpec(memory_space=pltpu.VMEM),
        scratch_shapes=[comm] * N_SUB + [sems] * (2 * N_SUB) + [
            pltpu.VMEM((k, n_per), jnp.float8_e5m2),
        ],
        compiler_params=pltpu.CompilerParams(
            collective_id=0, vmem_limit_bytes=100 * 1024 * 1024),
    )(x, w_mat, s)


# device time: 100210 ns/iter; 1.2253x vs baseline; 1.1343x over previous
import jax
import jax.numpy as jnp
from jax import lax
from jax.experimental import pallas as pl
from jax.experimental.pallas import tpu as pltpu

N_DEV = 8
N_PART = 3
PART_SIZES = (176, 168, 168)
PART_OFFS = (0, 176, 344)
DIM_ORDERS = ((0, 1, 2), (1, 2, 0), (2, 0, 1))


def kernel(x, w_mat, scale_x, scale_w):
    m_per, k = x.shape
    _, n_per = w_mat.shape

    s = (scale_x.reshape(-1)[:1] * scale_w.reshape(-1)[:1]).astype(jnp.float32)

    def body(x_ref, w_ref, s_ref, out_ref, *scratch):
        bufs = scratch[0:N_PART]
        sends = scratch[N_PART:2 * N_PART]
        recvs = scratch[2 * N_PART:3 * N_PART]
        w8_ref = scratch[3 * N_PART]

        my = lax.axis_index("i")

        zc = my // 4
        q = lax.rem(my, 4)
        xc = jnp.where((q == 1) | (q == 2), 1, 0)
        yc = jnp.where(q >= 2, 1, 0)
        coords = (xc, yc, zc)

        def make_id(c):
            return 4 * c[2] + 2 * c[1] + jnp.bitwise_xor(c[0], c[1])

        def flipped_id(dims):
            c = list(coords)
            for d in dims:
                c[d] = 1 - c[d]
            return make_id(c)

        neighbors = [flipped_id((d,)) for d in range(3)]

        barrier_sem = pltpu.get_barrier_semaphore()
        for nbr in neighbors:
            pl.semaphore_signal(
                barrier_sem, inc=1,
                device_id=(nbr,), device_id_type=pl.DeviceIdType.MESH,
            )
        pl.semaphore_wait(barrier_sem, 3)

        def subsets(p, step):
            used = DIM_ORDERS[p][:step]
            out = [()]
            for d in used:
                out = out + [m + (d,) for m in out]
            return out

        def xfer(p, step, j, g):
            d = DIM_ORDERS[p][step]
            return pltpu.make_async_remote_copy(
                src_ref=bufs[p].at[g],
                dst_ref=bufs[p].at[g],
                send_sem=sends[p].at[step, j],
                recv_sem=recvs[p].at[step, j],
                device_id=(neighbors[d],),
                device_id_type=pl.DeviceIdType.MESH,
            )

        def store(p, g):
            acc = lax.dot_general(
                bufs[p][g], w8_ref[...],
                (((1,), (0,)), ((), ())),
                preferred_element_type=jnp.float32,
            )
            out_ref[pl.ds(g * m_per + PART_OFFS[p], PART_SIZES[p]), :] = (
                acc * s_ref[0])

        for p in range(N_PART):
            bufs[p][my] = x_ref[
                PART_OFFS[p]:PART_OFFS[p] + PART_SIZES[p], :].astype(
                    jnp.float8_e4m3fn)
        for p in range(N_PART):
            xfer(p, 0, 0, my).start()
        w8_ref[...] = w_ref[...].astype(jnp.float8_e5m2)
        for p in range(N_PART):
            store(p, my)

        for step in range(3):
            for p in range(N_PART):
                d = DIM_ORDERS[p][step]
                for j, mask in enumerate(subsets(p, step)):
                    g_recv = flipped_id(mask + (d,))
                    xfer(p, step, j, g_recv).wait_recv()
            if step < 2:
                for p in range(N_PART):
                    d = DIM_ORDERS[p][step]
                    for j, mask in enumerate(subsets(p, step + 1)):
                        xfer(p, step + 1, j, flipped_id(mask)).start()
            for p in range(N_PART):
                d = DIM_ORDERS[p][step]
                for mask in subsets(p, step):
                    store(p, flipped_id(mask + (d,)))

        for p in range(N_PART):
            for step in range(3):
                for j in range(2 ** step):
                    xfer(p, step, j, my).wait_send()

    sems = pltpu.SemaphoreType.DMA((3, 4))
    return pl.pallas_call(
        body,
        out_shape=jax.ShapeDtypeStruct((N_DEV * m_per, n_per), jnp.float32),
        in_specs=[
            pl.BlockSpec(memory_space=pltpu.VMEM),
            pl.BlockSpec(memory_space=pltpu.VMEM),
            pl.BlockSpec(memory_space=pltpu.SMEM),
        ],
        out_specs=pl.BlockSpec(memory_space=pltpu.VMEM),
        scratch_shapes=[
            pltpu.VMEM((N_DEV, PART_SIZES[p], k), jnp.float8_e4m3fn)
            for p in range(N_PART)
        ] + [sems] * (2 * N_PART) + [
            pltpu.VMEM((k, n_per), jnp.float8_e5m2),
        ],
        compiler_params=pltpu.CompilerParams(
            collective_id=0, vmem_limit_bytes=100 * 1024 * 1024),
    )(x, w_mat, s)


# device time: 96920 ns/iter; 1.2669x vs baseline; 1.0339x over previous
import jax
import jax.numpy as jnp
from jax import lax
from jax.experimental import pallas as pl
from jax.experimental.pallas import tpu as pltpu

N_DEV = 8
N_PART = 3
PART_SIZES = (176, 168, 168)
PART_OFFS = (0, 176, 344)
DIM_ORDERS = ((0, 1, 2), (1, 2, 0), (2, 0, 1))


def kernel(x, w_mat, scale_x, scale_w):
    m_per, k = x.shape
    _, n_per = w_mat.shape

    s = (scale_x.reshape(-1)[:1] * scale_w.reshape(-1)[:1]).astype(jnp.float32)

    def body(x_ref, w_ref, s_ref, out_ref, *scratch):
        bufs = scratch[0:N_PART]
        sends = scratch[N_PART:2 * N_PART]
        recvs = scratch[2 * N_PART:3 * N_PART]
        w8_ref = scratch[3 * N_PART]

        my = lax.axis_index("i")

        zc = my // 4
        q = lax.rem(my, 4)
        xc = jnp.where((q == 1) | (q == 2), 1, 0)
        yc = jnp.where(q >= 2, 1, 0)
        coords = (xc, yc, zc)

        def make_id(c):
            return 4 * c[2] + 2 * c[1] + jnp.bitwise_xor(c[0], c[1])

        def flipped_id(dims):
            c = list(coords)
            for d in dims:
                c[d] = 1 - c[d]
            return make_id(c)

        neighbors = [flipped_id((d,)) for d in range(3)]

        barrier_sem = pltpu.get_barrier_semaphore()
        for nbr in neighbors:
            pl.semaphore_signal(
                barrier_sem, inc=1,
                device_id=(nbr,), device_id_type=pl.DeviceIdType.MESH,
            )
        pl.semaphore_wait(barrier_sem, 3)

        def subsets(p, step):
            used = DIM_ORDERS[p][:step]
            out = [()]
            for d in used:
                out = out + [m + (d,) for m in out]
            return out

        def xfer(p, step, j, g):
            d = DIM_ORDERS[p][step]
            return pltpu.make_async_remote_copy(
                src_ref=bufs[p].at[g],
                dst_ref=bufs[p].at[g],
                send_sem=sends[p].at[step, j],
                recv_sem=recvs[p].at[step, j],
                device_id=(neighbors[d],),
                device_id_type=pl.DeviceIdType.MESH,
            )

        def store(p, g):
            acc = lax.dot_general(
                bufs[p][g], w8_ref[...],
                (((1,), (0,)), ((), ())),
                preferred_element_type=jnp.float32,
            )
            out_ref[pl.ds(g * m_per + PART_OFFS[p], PART_SIZES[p]), :] = (
                acc * s_ref[0])

        def jidx(p, mask):
            return sum(
                2 ** i for i, d in enumerate(DIM_ORDERS[p]) if d in mask)

        for p in range(N_PART):
            bufs[p][my] = x_ref[
                PART_OFFS[p]:PART_OFFS[p] + PART_SIZES[p], :].astype(
                    jnp.float8_e4m3fn)
        for step in range(3):
            for p in range(N_PART):
                xfer(p, step, 0, my).start()
        w8_ref[...] = w_ref[...].astype(jnp.float8_e5m2)
        for p in range(N_PART):
            store(p, my)

        for step in range(3):
            for p in range(N_PART):
                d = DIM_ORDERS[p][step]
                for mask in subsets(p, step):
                    newmask = mask + (d,)
                    g = flipped_id(newmask)
                    xfer(p, step, jidx(p, mask), g).wait_recv()
                    for step2 in range(step + 1, 3):
                        xfer(p, step2, jidx(p, newmask), g).start()
                    store(p, g)

        for p in range(N_PART):
            for step in range(3):
                for j in range(2 ** step):
                    xfer(p, step, j, my).wait_send()

    sems = pltpu.SemaphoreType.DMA((3, 4))
    return pl.pallas_call(
        body,
        out_shape=jax.ShapeDtypeStruct((N_DEV * m_per, n_per), jnp.float32),
        in_specs=[
            pl.BlockSpec(memory_space=pltpu.VMEM),
            pl.BlockSpec(memory_space=pltpu.VMEM),
            pl.BlockSpec(memory_space=pltpu.SMEM),
        ],
        out_specs=pl.BlockSpec(memory_space=pltpu.VMEM),
        scratch_shapes=[
            pltpu.VMEM((N_DEV, PART_SIZES[p], k), jnp.float8_e4m3fn)
            for p in range(N_PART)
        ] + [sems] * (2 * N_PART) + [
            pltpu.VMEM((k, n_per), jnp.float8_e5m2),
        ],
        compiler_params=pltpu.CompilerParams(
            collective_id=0, vmem_limit_bytes=100 * 1024 * 1024),
    )(x, w_mat, s)
